# Initial kernel scaffold; baseline (speedup 1.0000x reference)
import jax
import jax.numpy as jnp
from jax.experimental import pallas as pl


def kernel(features, edge_index, etypes, lin_W0, lin_b0, gru_Wih0, gru_Whh0, gru_bih0, gru_bhh0, lin_W1, lin_b1, gru_Wih1, gru_Whh1, gru_bih1, gru_bhh1, gate_w, gate_b, out_w, out_b):
    raise NotImplementedError("write your pallas kernel here")


# speedup vs baseline: 5.0609x; 5.0609x over previous
import functools

import jax
import jax.numpy as jnp
from jax import lax
from jax.experimental import pallas as pl
from jax.experimental.pallas import tpu as pltpu
from jax.experimental.pallas import tpu_sc as plsc

N = 10000
E = 160000
D = 256
NET = 4
NSTEPS = 5
NCLS = 2

NSUB = 16
NCORE = 2
CHUNK = 128
CH = 80
SB = 16
EDGES_PER_SUB = CHUNK * CH
E_PAD = EDGES_PER_SUB * NSUB
ACC_ROWS = 10112
ZROWS = ACC_ROWS // NSUB
OROWS = ZROWS
TROWS = NCORE * NET * N
HALF = 128


def _sc_body(table, gidx, didx, zeros, a_out,
             gslab, dslab, buf0, buf1, sem0, sem1, acc):
    cid = lax.axis_index("c")
    sid = lax.axis_index("s")
    pltpu.sync_copy(gidx.at[cid, sid], gslab)
    pltpu.sync_copy(didx.at[sid], dslab)
    pltpu.sync_copy(zeros.at[pl.ds(sid * ZROWS, ZROWS)],
                    acc.at[pl.ds(sid * ZROWS, ZROWS)])
    plsc.subcore_barrier()

    for sb in range(CH // SB):
        pltpu.sync_copy(gidx.at[cid, sid * (CH // SB) + sb], gslab)
        pltpu.sync_copy(didx.at[sid * (CH // SB) + sb], dslab)
        pltpu.async_copy(table.at[gslab.at[0]], buf0, sem0)

        @pl.loop(0, SB, step=2)
        def _chunks(j):
            pltpu.async_copy(table.at[gslab.at[j + 1]], buf1, sem1)
            pltpu.make_async_copy(table.at[gslab.at[j]], buf0, sem0).wait()
            pltpu.sync_copy(buf0, acc.at[dslab.at[j]], add=True)

            @pl.when(j + 2 < SB)
            def _():
                pltpu.async_copy(table.at[gslab.at[j + 2]], buf0, sem0)

            pltpu.make_async_copy(table.at[gslab.at[j + 1]], buf1, sem1).wait()
            pltpu.sync_copy(buf1, acc.at[dslab.at[j + 1]], add=True)

    plsc.subcore_barrier()
    pltpu.sync_copy(acc.at[pl.ds(sid * OROWS, OROWS)], a_out.at[cid, sid])


@functools.cache
def _sc_aggregate_call():
  return pl.kernel(
    _sc_body,
    out_type=jax.ShapeDtypeStruct((NCORE, NSUB, OROWS, HALF), jnp.float32),
    mesh=plsc.VectorSubcoreMesh(core_axis_name="c", subcore_axis_name="s"),
    scratch_types=[
        pltpu.VMEM((SB, CHUNK), jnp.int32),
        pltpu.VMEM((SB, CHUNK), jnp.int32),
        pltpu.VMEM((CHUNK, HALF), jnp.float32),
        pltpu.VMEM((CHUNK, HALF), jnp.float32),
        pltpu.SemaphoreType.DMA,
        pltpu.SemaphoreType.DMA,
        pltpu.VMEM_SHARED((ACC_ROWS, HALF), jnp.float32),
    ],
  )


BLK = 2000
NBLK = N // BLK


def _write_table(tab_ref, z):
    for t in range(NET):
        zt = z[:, t * D:(t + 1) * D]
        tab_ref[0, t] = zt[:, :HALF]
        tab_ref[1, t] = zt[:, HALF:]


def _table_body(h_ref, wcat_ref, bcat_ref, tab_ref):
    z = jnp.dot(h_ref[...], wcat_ref[...],
                preferred_element_type=jnp.float32) + bcat_ref[...]
    _write_table(tab_ref, z)


_table_call = pl.pallas_call(
    _table_body,
    grid=(NBLK,),
    in_specs=[
        pl.BlockSpec((BLK, D), lambda i: (i, 0)),
        pl.BlockSpec((D, NET * D), lambda i: (0, 0)),
        pl.BlockSpec((1, NET * D), lambda i: (0, 0)),
    ],
    out_specs=pl.BlockSpec((NCORE, NET, BLK, HALF), lambda i: (0, 0, i, 0)),
    out_shape=jax.ShapeDtypeStruct((NCORE, NET, N, HALF), jnp.float32),
)


def _gru_core(a_ref, h_ref, wih_ref, whh_ref, bih_ref, bhh_ref, relu):
    ab = jnp.concatenate([a_ref[0], a_ref[1]], axis=-1)
    hb = h_ref[...]
    gi = jnp.dot(ab, wih_ref[...], preferred_element_type=jnp.float32) + bih_ref[...]
    gh = jnp.dot(hb, whh_ref[...], preferred_element_type=jnp.float32) + bhh_ref[...]
    r = jax.nn.sigmoid(gi[:, :D] + gh[:, :D])
    z = jax.nn.sigmoid(gi[:, D:2 * D] + gh[:, D:2 * D])
    n = jnp.tanh(gi[:, 2 * D:] + r * gh[:, 2 * D:])
    hn = (1.0 - z) * n + z * hb
    if relu:
        hn = jnp.maximum(hn, 0.0)
    return hn


def _gru_table_body(a_ref, h_ref, wih_ref, whh_ref, bih_ref, bhh_ref,
                    wcat_ref, bcat_ref, h_out, tab_ref, *, relu):
    hn = _gru_core(a_ref, h_ref, wih_ref, whh_ref, bih_ref, bhh_ref, relu)
    h_out[...] = hn
    z = jnp.dot(hn, wcat_ref[...],
                preferred_element_type=jnp.float32) + bcat_ref[...]
    _write_table(tab_ref, z)


def _gru_only_body(a_ref, h_ref, wih_ref, whh_ref, bih_ref, bhh_ref,
                   h_out, *, relu):
    h_out[...] = _gru_core(a_ref, h_ref, wih_ref, whh_ref, bih_ref, bhh_ref, relu)


def _make_gru_table_call(relu):
    return pl.pallas_call(
        functools.partial(_gru_table_body, relu=relu),
        grid=(NBLK,),
        in_specs=[
            pl.BlockSpec((NCORE, BLK, HALF), lambda i: (0, i, 0)),
            pl.BlockSpec((BLK, D), lambda i: (i, 0)),
            pl.BlockSpec((D, 3 * D), lambda i: (0, 0)),
            pl.BlockSpec((D, 3 * D), lambda i: (0, 0)),
            pl.BlockSpec((1, 3 * D), lambda i: (0, 0)),
            pl.BlockSpec((1, 3 * D), lambda i: (0, 0)),
            pl.BlockSpec((D, NET * D), lambda i: (0, 0)),
            pl.BlockSpec((1, NET * D), lambda i: (0, 0)),
        ],
        out_specs=[
            pl.BlockSpec((BLK, D), lambda i: (i, 0)),
            pl.BlockSpec((NCORE, NET, BLK, HALF), lambda i: (0, 0, i, 0)),
        ],
        out_shape=[
            jax.ShapeDtypeStruct((N, D), jnp.float32),
            jax.ShapeDtypeStruct((NCORE, NET, N, HALF), jnp.float32),
        ],
    )


_gru_table = _make_gru_table_call(relu=False)
_gru_relu_table = _make_gru_table_call(relu=True)

_gru_relu_only = pl.pallas_call(
    functools.partial(_gru_only_body, relu=True),
    grid=(NBLK,),
    in_specs=[
        pl.BlockSpec((NCORE, BLK, HALF), lambda i: (0, i, 0)),
        pl.BlockSpec((BLK, D), lambda i: (i, 0)),
        pl.BlockSpec((D, 3 * D), lambda i: (0, 0)),
        pl.BlockSpec((D, 3 * D), lambda i: (0, 0)),
        pl.BlockSpec((1, 3 * D), lambda i: (0, 0)),
        pl.BlockSpec((1, 3 * D), lambda i: (0, 0)),
    ],
    out_specs=pl.BlockSpec((BLK, D), lambda i: (i, 0)),
    out_shape=jax.ShapeDtypeStruct((N, D), jnp.float32),
)


def _readout_body(h_ref, gw_ref, gb_ref, ow_ref, ob_ref, out_ref):
    hb = h_ref[...]
    s = jnp.sum(hb * gw_ref[...], axis=1, keepdims=True) + gb_ref[...]
    m = jnp.max(s)
    e = jnp.exp(s - m)
    w = e / jnp.sum(e)
    r = jnp.sum(w * hb, axis=0, keepdims=True)
    logits = jnp.sum(r * ow_ref[...], axis=1)
    out_ref[...] = logits[None, :] + ob_ref[...]


_readout = pl.pallas_call(
    _readout_body,
    in_specs=[
        pl.BlockSpec((N, D), lambda: (0, 0)),
        pl.BlockSpec((1, D), lambda: (0, 0)),
        pl.BlockSpec((1, 1), lambda: (0, 0)),
        pl.BlockSpec((NCLS, D), lambda: (0, 0)),
        pl.BlockSpec((1, NCLS), lambda: (0, 0)),
    ],
    out_specs=pl.BlockSpec((1, NCLS), lambda: (0, 0)),
    out_shape=jax.ShapeDtypeStruct((1, NCLS), jnp.float32),
)


def kernel(features, edge_index, etypes, lin_W0, lin_b0, gru_Wih0, gru_Whh0,
           gru_bih0, gru_bhh0, lin_W1, lin_b1, gru_Wih1, gru_Whh1, gru_bih1,
           gru_bhh1, gate_w, gate_b, out_w, out_b):
    src = edge_index[0]
    dst = edge_index[1]

    pad = E_PAD - E
    fi = etypes * N + src
    gfi = jnp.concatenate([fi, jnp.zeros((pad,), jnp.int32)])
    gfi = gfi.reshape(NSUB * (CH // SB), SB, CHUNK)
    gidx = jnp.stack([gfi, gfi + NET * N])
    didx = jnp.concatenate([dst, jnp.full((pad,), N, jnp.int32)])
    didx = didx.reshape(NSUB * (CH // SB), SB, CHUNK)
    zeros = jnp.zeros((ACC_ROWS, HALF), jnp.float32)

    wcat0 = jnp.concatenate([lin_W0[t].T for t in range(NET)], axis=1)
    bcat0 = lin_b0.reshape(1, NET * D)
    wcat1 = jnp.concatenate([lin_W1[t].T for t in range(NET)], axis=1)
    bcat1 = lin_b1.reshape(1, NET * D)
    wih0, whh0 = gru_Wih0.T, gru_Whh0.T
    bih0, bhh0 = gru_bih0.reshape(1, 3 * D), gru_bhh0.reshape(1, 3 * D)
    wih1, whh1 = gru_Wih1.T, gru_Whh1.T
    bih1, bhh1 = gru_bih1.reshape(1, 3 * D), gru_bhh1.reshape(1, 3 * D)

    h = features
    table = _table_call(h, wcat0, bcat0)
    for step in range(NSTEPS):
        a = _sc_aggregate_call()(table.reshape(TROWS, HALF), gidx, didx,
                                 zeros).reshape(NCORE, ACC_ROWS, HALF)
        if step < NSTEPS - 1:
            h, table = _gru_table(a, h, wih0, whh0, bih0, bhh0, wcat0, bcat0)
        else:
            h, table = _gru_relu_table(a, h, wih0, whh0, bih0, bhh0,
                                       wcat1, bcat1)
    for step in range(NSTEPS):
        a = _sc_aggregate_call()(table.reshape(TROWS, HALF), gidx, didx,
                                 zeros).reshape(NCORE, ACC_ROWS, HALF)
        if step < NSTEPS - 1:
            h, table = _gru_table(a, h, wih1, whh1, bih1, bhh1, wcat1, bcat1)
        else:
            h = _gru_relu_only(a, h, wih1, whh1, bih1, bhh1)

    return _readout(h, gate_w, gate_b.reshape(1, 1), out_w,
                    out_b.reshape(1, NCLS))
